# baseline (speedup 1.0000x reference)
import functools

import jax
import jax.numpy as jnp
from jax import lax
from jax.experimental import pallas as pl
from jax.experimental.pallas import tpu as pltpu
from jax.experimental.pallas import tpu_sc as plsc

_SPATIAL_SCALE = 0.125
_P = 7
_TRANS_STD = 0.1
_B, _C, _H, _W = 2, 64, 256, 256
_N = 512
_NC, _NS, _L = 2, 16, 16
_NW = _NC * _NS
_RPW = _N // _NW
_CELLS = _P * _P
_HROWS = 8 * 2 * _L
_GCHUNK = 128


def _round_half_even(x):
    ri = x.astype(jnp.int32)
    rf = ri.astype(jnp.float32)
    frac = x - rf
    odd = (ri & 1) == 1
    up = (frac > 0.5) | ((frac == 0.5) & odd)
    return jnp.where(up, rf + 1.0, rf)


def _dpool_body(table, roisf, offf, out, rvm, offvm, wb0, wb1, idxb0, idxb1,
                invbuf, acc_blk, rows0, rows1, out_stage, sem0, sem1):
    cid = lax.axis_index("c")
    sid = lax.axis_index("s")
    wid = sid * _NC + cid
    rbase = wid * _RPW

    for j in range(5):
        pltpu.sync_copy(roisf.at[pl.ds(j * _N + rbase, _L)], rvm.at[j])
    pltpu.sync_copy(offf.at[pl.ds(rbase * 2 * _CELLS, _RPW * 2 * _CELLS)], offvm)

    iota16 = lax.broadcasted_iota(jnp.int32, (_L,), 0)
    off_row0 = iota16 * (2 * _CELLS)

    rb = rvm[0].astype(jnp.int32)
    x1 = _round_half_even(rvm[1])
    y1 = _round_half_even(rvm[2])
    x2 = _round_half_even(rvm[3])
    y2 = _round_half_even(rvm[4])
    rsw = x1 * _SPATIAL_SCALE - 0.5
    rsh = y1 * _SPATIAL_SCALE - 0.5
    rew = (x2 + 1.0) * _SPATIAL_SCALE - 0.5
    reh = (y2 + 1.0) * _SPATIAL_SCALE - 0.5
    roi_w = jnp.maximum(rew - rsw, 0.1)
    roi_h = jnp.maximum(reh - rsh, 0.1)
    bin_w = roi_w / float(_P)
    bin_h = roi_h / float(_P)
    sub_w = bin_w / 4.0
    sub_h = bin_h / 4.0
    base_i = rb * (_H * _W)

    scat_cols = [(jnp.int32((j // 2) * 32 + (j % 2)) + iota16 * 2) * _CELLS
                 for j in range(4)]

    def coords(cell, half, idxb, wbufr):
        ph = cell // _P
        pw = cell - ph * _P
        csplat = jnp.full((_L,), cell, jnp.int32)
        tx = plsc.load_gather(offvm, [off_row0 + csplat]) * _TRANS_STD
        ty = plsc.load_gather(offvm, [off_row0 + (csplat + _CELLS)]) * _TRANS_STD
        hst = ph.astype(jnp.float32) * bin_h + rsh + ty * roi_h
        wst = pw.astype(jnp.float32) * bin_w + rsw + tx * roi_w
        cnt = jnp.zeros((_L,), jnp.float32)
        for sl in range(8):
            s = half * 8 + sl
            ih, iw = s // 4, s % 4
            w_ = wst + float(iw) * sub_w
            h_ = hst + float(ih) * sub_h
            mask = (w_ > -0.5) & (w_ < _W - 0.5) & (h_ > -0.5) & (h_ < _H - 0.5)
            mf = jnp.where(mask, 1.0, 0.0)
            cnt = cnt + mf
            wc = jnp.clip(w_, 0.0, _W - 1.0)
            hc = jnp.clip(h_, 0.0, _H - 1.0)
            w0 = wc.astype(jnp.int32)
            h0 = hc.astype(jnp.int32)
            w1 = jnp.minimum(w0 + 1, _W - 1)
            h1 = jnp.minimum(h0 + 1, _H - 1)
            lw = wc - w0.astype(jnp.float32)
            lh = hc - h0.astype(jnp.float32)
            hw = 1.0 - lw
            hh = 1.0 - lh
            del w1
            idxs = (base_i + h0 * _W + w0, base_i + h1 * _W + w0)
            ws = (hh * hw * mf, hh * lw * mf, lh * hw * mf, lh * lw * mf)
            for hc in range(2):
                k2 = sl * 2 + hc
                idxb[pl.ds(k2 * _L, _L)] = idxs[hc]
            for ci in range(4):
                k = sl * 4 + ci
                wbufr[pl.ds(k * _L, _L)] = ws[ci]
        return cnt

    def fire(idxb, rows, sem):
        for g in range(_HROWS // _GCHUNK):
            pltpu.async_copy(
                table.at[idxb.at[pl.ds(g * _GCHUNK, _GCHUNK)]],
                rows.at[pl.ds(g * _GCHUNK, _GCHUNK)], sem)

    def drain(rows, sem):
        pltpu.make_async_copy(table.at[pl.ds(0, _HROWS)], rows, sem).wait()

    ksplats = [jnp.full((_L,), kk, jnp.int32) for kk in range(16)]

    def accum(rows, wbufr, r, rsplat, acc):
        w_lo = plsc.load_gather(wbufr, [iota16 * _L + rsplat])
        w_hi = plsc.load_gather(wbufr, [iota16 * _L + rsplat + 16 * _L])
        for g in range(4):
            a = [jnp.zeros((32,), jnp.bfloat16) for _ in range(2)]
            for pp in range(4):
                p = g * 4 + pp
                wsrc = w_lo if p < 8 else w_hi
                wl = jnp.take_along_axis(wsrc, ksplats[(2 * p) % 16], axis=0)
                wr = jnp.take_along_axis(wsrc, ksplats[(2 * p + 1) % 16],
                                         axis=0)
                wbl = plsc.pack(wl, wl, format=plsc.PackFormat.INTERLEAVED)
                wbr = plsc.pack(wr, wr, format=plsc.PackFormat.INTERLEAVED)
                row = p * _L + r
                for h in range(2):
                    a[h] = (a[h] + wbl * rows[row, pl.ds(h * 32, 32)]
                            + wbr * rows[row, pl.ds(64 + h * 32, 32)])
            for h in range(2):
                ev, od = plsc.unpack(a[h], format=plsc.PackFormat.INTERLEAVED)
                acc[2 * h] = acc[2 * h] + ev
                acc[2 * h + 1] = acc[2 * h + 1] + od
        return acc

    def accum_h0(r, _):
        rsplat = jnp.full((_L,), r, jnp.int32)
        acc = [jnp.zeros((_L,), jnp.float32) for _ in range(4)]
        acc = accum(rows0, wb0, r, rsplat, acc)
        for j in range(4):
            acc_blk[r, pl.ds(j * _L, _L)] = acc[j]
        return 0

    def make_accum_h1(cell):
        def accum_h1(r, _):
            rsplat = jnp.full((_L,), r, jnp.int32)
            invs = plsc.load_gather(invbuf, [rsplat])
            acc = [acc_blk[r, pl.ds(j * _L, _L)] for j in range(4)]
            acc = accum(rows1, wb1, r, rsplat, acc)
            sbase = jnp.full((_L,), r * (_C * _CELLS) + cell, jnp.int32)
            for j in range(4):
                plsc.store_scatter(out_stage, [sbase + scat_cols[j]],
                                   acc[j] * invs)
            return 0
        return accum_h1

    cnt0_init = coords(jnp.int32(0), 0, idxb0, wb0)
    fire(idxb0, rows0, sem0)

    def cell_body(cell, cnt0):
        cnt1 = coords(cell, 1, idxb1, wb1)
        fire(idxb1, rows1, sem1)
        invbuf[...] = 1.0 / jnp.maximum(cnt0 + cnt1, 1.0)
        drain(rows0, sem0)
        lax.fori_loop(0, _RPW, accum_h0, 0)
        cnext = jnp.minimum(cell + 1, _CELLS - 1)
        cnt0_new = coords(cnext, 0, idxb0, wb0)

        @pl.when(cell < _CELLS - 1)
        def _():
            fire(idxb0, rows0, sem0)

        drain(rows1, sem1)
        lax.fori_loop(0, _RPW, make_accum_h1(cell), 0)
        return cnt0_new

    lax.fori_loop(0, _CELLS, cell_body, cnt0_init)
    pltpu.sync_copy(out_stage,
                    out.at[pl.ds(rbase * _C * _CELLS, _RPW * _C * _CELLS)])


_TBLK = 8


def _tr_body(x_ref, o_ref):
    x = x_ref[0].reshape(_C, _TBLK * _W)
    y = x.T.astype(jnp.bfloat16)
    ys = jnp.concatenate([y[1:], y[:1]], axis=0)
    o_ref[...] = jnp.concatenate([y, ys], axis=1)


def _to_table(data):
    blk = _TBLK * _W
    return pl.pallas_call(
        _tr_body,
        grid=(_B, _H // _TBLK),
        in_specs=[pl.BlockSpec((1, _C, _TBLK, _W), lambda b, h: (b, 0, h, 0))],
        out_specs=pl.BlockSpec((blk, 2 * _C),
                               lambda b, h: (b * (_H // _TBLK) + h, 0)),
        out_shape=jax.ShapeDtypeStruct((_B * _H * _W, 2 * _C), jnp.bfloat16),
    )(data)


@jax.jit
def _dpool(table, roisf, offf):
    mesh = plsc.VectorSubcoreMesh(core_axis_name="c", subcore_axis_name="s")
    run = functools.partial(
        pl.kernel,
        mesh=mesh,
        compiler_params=pltpu.CompilerParams(
            needs_layout_passes=False, use_tc_tiling_on_sc=False),
        out_type=jax.ShapeDtypeStruct((_N * _C * _CELLS,), jnp.float32),
        scratch_types=[
            pltpu.VMEM((5, _L), jnp.float32),
            pltpu.VMEM((_RPW * 2 * _CELLS,), jnp.float32),
            pltpu.VMEM((2 * _HROWS,), jnp.float32),
            pltpu.VMEM((2 * _HROWS,), jnp.float32),
            pltpu.VMEM((_HROWS,), jnp.int32),
            pltpu.VMEM((_HROWS,), jnp.int32),
            pltpu.VMEM((_L,), jnp.float32),
            pltpu.VMEM((_RPW, _C), jnp.float32),
            pltpu.VMEM((_HROWS, 2 * _C), jnp.bfloat16),
            pltpu.VMEM((_HROWS, 2 * _C), jnp.bfloat16),
            pltpu.VMEM((_RPW * _C * _CELLS,), jnp.float32),
            pltpu.SemaphoreType.DMA,
            pltpu.SemaphoreType.DMA,
        ],
    )(_dpool_body)
    return run(table, roisf, offf)


def kernel(data, rois, offset):
    table = _to_table(data)
    roisf = jnp.transpose(rois, (1, 0)).reshape(-1)
    offf = offset.reshape(-1)
    out = _dpool(table, roisf, offf)
    return out.reshape(_N, _C, _P, _P)


# speedup vs baseline: 1.0798x; 1.0798x over previous
import functools

import jax
import jax.numpy as jnp
from jax import lax
from jax.experimental import pallas as pl
from jax.experimental.pallas import tpu as pltpu
from jax.experimental.pallas import tpu_sc as plsc

_SPATIAL_SCALE = 0.125
_P = 7
_TRANS_STD = 0.1
_B, _C, _H, _W = 2, 64, 256, 256
_N = 512
_NC, _NS, _L = 2, 16, 16
_NW = _NC * _NS
_RPW = _N // _NW
_CELLS = _P * _P
_HROWS = 8 * 4 * _L
_GCHUNK = 128


def _round_half_even(x):
    ri = x.astype(jnp.int32)
    rf = ri.astype(jnp.float32)
    frac = x - rf
    odd = (ri & 1) == 1
    up = (frac > 0.5) | ((frac == 0.5) & odd)
    return jnp.where(up, rf + 1.0, rf)


def _dpool_body(table, roisf, offf, out, rvm, offvm, wb0, wb1, idxb0, idxb1,
                invbuf, acc_blk, rows0, rows1, out_stage, sem0, sem1):
    cid = lax.axis_index("c")
    sid = lax.axis_index("s")
    wid = sid * _NC + cid
    rbase = wid * _RPW

    for j in range(5):
        pltpu.sync_copy(roisf.at[pl.ds(j * _N + rbase, _L)], rvm.at[j])
    pltpu.sync_copy(offf.at[pl.ds(rbase * 2 * _CELLS, _RPW * 2 * _CELLS)], offvm)

    iota16 = lax.broadcasted_iota(jnp.int32, (_L,), 0)
    off_row0 = iota16 * (2 * _CELLS)

    rb = rvm[0].astype(jnp.int32)
    x1 = _round_half_even(rvm[1])
    y1 = _round_half_even(rvm[2])
    x2 = _round_half_even(rvm[3])
    y2 = _round_half_even(rvm[4])
    rsw = x1 * _SPATIAL_SCALE - 0.5
    rsh = y1 * _SPATIAL_SCALE - 0.5
    rew = (x2 + 1.0) * _SPATIAL_SCALE - 0.5
    reh = (y2 + 1.0) * _SPATIAL_SCALE - 0.5
    roi_w = jnp.maximum(rew - rsw, 0.1)
    roi_h = jnp.maximum(reh - rsh, 0.1)
    bin_w = roi_w / float(_P)
    bin_h = roi_h / float(_P)
    sub_w = bin_w / 4.0
    sub_h = bin_h / 4.0
    base_i = rb * (_H * _W)

    scat_cols = [(jnp.int32((j // 2) * 32 + (j % 2)) + iota16 * 2) * _CELLS
                 for j in range(4)]

    def coords(cell, half, idxb, wbufr):
        ph = cell // _P
        pw = cell - ph * _P
        csplat = jnp.full((_L,), cell, jnp.int32)
        tx = plsc.load_gather(offvm, [off_row0 + csplat]) * _TRANS_STD
        ty = plsc.load_gather(offvm, [off_row0 + (csplat + _CELLS)]) * _TRANS_STD
        hst = ph.astype(jnp.float32) * bin_h + rsh + ty * roi_h
        wst = pw.astype(jnp.float32) * bin_w + rsw + tx * roi_w
        cnt = jnp.zeros((_L,), jnp.float32)
        for sl in range(8):
            s = half * 8 + sl
            ih, iw = s // 4, s % 4
            w_ = wst + float(iw) * sub_w
            h_ = hst + float(ih) * sub_h
            mask = (w_ > -0.5) & (w_ < _W - 0.5) & (h_ > -0.5) & (h_ < _H - 0.5)
            mf = jnp.where(mask, 1.0, 0.0)
            cnt = cnt + mf
            wc = jnp.clip(w_, 0.0, _W - 1.0)
            hc = jnp.clip(h_, 0.0, _H - 1.0)
            w0 = wc.astype(jnp.int32)
            h0 = hc.astype(jnp.int32)
            w1 = jnp.minimum(w0 + 1, _W - 1)
            h1 = jnp.minimum(h0 + 1, _H - 1)
            lw = wc - w0.astype(jnp.float32)
            lh = hc - h0.astype(jnp.float32)
            hw = 1.0 - lw
            hh = 1.0 - lh
            r0 = base_i + h0 * _W
            r1 = base_i + h1 * _W
            idxs = (r0 + w0, r0 + w1, r1 + w0, r1 + w1)
            ws = (hh * hw * mf, hh * lw * mf, lh * hw * mf, lh * lw * mf)
            for ci in range(4):
                k = sl * 4 + ci
                idxb[pl.ds(k * _L, _L)] = idxs[ci]
                wbufr[pl.ds(k * _L, _L)] = ws[ci]
        return cnt

    def fire(idxb, rows, sem):
        for g in range(_HROWS // _GCHUNK):
            pltpu.async_copy(
                table.at[idxb.at[pl.ds(g * _GCHUNK, _GCHUNK)]],
                rows.at[pl.ds(g * _GCHUNK, _GCHUNK)], sem)

    def drain(rows, sem):
        pltpu.make_async_copy(table.at[pl.ds(0, _HROWS)], rows, sem).wait()

    ksplats = [jnp.full((_L,), kk, jnp.int32) for kk in range(16)]

    def accum(rows, wbufr, r, rsplat, acc):
        w_lo = plsc.load_gather(wbufr, [iota16 * _L + rsplat])
        w_hi = plsc.load_gather(wbufr, [iota16 * _L + rsplat + 16 * _L])
        for g in range(4):
            a = [jnp.zeros((32,), jnp.bfloat16) for _ in range(2)]
            for kk in range(8):
                k = g * 8 + kk
                wsrc = w_lo if k < 16 else w_hi
                wv = jnp.take_along_axis(wsrc, ksplats[k % 16], axis=0)
                wb16 = plsc.pack(wv, wv, format=plsc.PackFormat.INTERLEAVED)
                row = k * _L + r
                for h in range(2):
                    a[h] = a[h] + wb16 * rows[row, pl.ds(h * 32, 32)]
            for h in range(2):
                ev, od = plsc.unpack(a[h], format=plsc.PackFormat.INTERLEAVED)
                acc[2 * h] = acc[2 * h] + ev
                acc[2 * h + 1] = acc[2 * h + 1] + od
        return acc

    def accum_h0(rh, _):
        for u in range(2):
            r = rh * 2 + u
            rsplat = jnp.full((_L,), r, jnp.int32)
            acc = [jnp.zeros((_L,), jnp.float32) for _ in range(4)]
            acc = accum(rows0, wb0, r, rsplat, acc)
            for j in range(4):
                acc_blk[r, pl.ds(j * _L, _L)] = acc[j]
        return 0

    def make_accum_h1(cell):
        def accum_h1(rh, _):
            for u in range(2):
                r = rh * 2 + u
                rsplat = jnp.full((_L,), r, jnp.int32)
                invs = plsc.load_gather(invbuf, [rsplat])
                acc = [acc_blk[r, pl.ds(j * _L, _L)] for j in range(4)]
                acc = accum(rows1, wb1, r, rsplat, acc)
                sbase = jnp.full((_L,), r * (_C * _CELLS) + cell, jnp.int32)
                for j in range(4):
                    plsc.store_scatter(out_stage, [sbase + scat_cols[j]],
                                       acc[j] * invs)
            return 0
        return accum_h1

    cnt0_init = coords(jnp.int32(0), 0, idxb0, wb0)
    fire(idxb0, rows0, sem0)

    def cell_body(cell, cnt0):
        cnt1 = coords(cell, 1, idxb1, wb1)
        fire(idxb1, rows1, sem1)
        invbuf[...] = 1.0 / jnp.maximum(cnt0 + cnt1, 1.0)
        drain(rows0, sem0)
        lax.fori_loop(0, _RPW // 2, accum_h0, 0)
        cnext = jnp.minimum(cell + 1, _CELLS - 1)
        cnt0_new = coords(cnext, 0, idxb0, wb0)

        @pl.when(cell < _CELLS - 1)
        def _():
            fire(idxb0, rows0, sem0)

        drain(rows1, sem1)
        lax.fori_loop(0, _RPW // 2, make_accum_h1(cell), 0)
        return cnt0_new

    lax.fori_loop(0, _CELLS, cell_body, cnt0_init)
    pltpu.sync_copy(out_stage,
                    out.at[pl.ds(rbase * _C * _CELLS, _RPW * _C * _CELLS)])


_TBLK = 8


def _tr_body(x_ref, o_ref):
    x = x_ref[0].reshape(_C, _TBLK * _W)
    o_ref[0] = x.T.reshape(_TBLK, _W, _C).astype(jnp.bfloat16)


def _to_table(data):
    out = pl.pallas_call(
        _tr_body,
        grid=(_B, _H // _TBLK),
        in_specs=[pl.BlockSpec((1, _C, _TBLK, _W), lambda b, h: (b, 0, h, 0))],
        out_specs=pl.BlockSpec((1, _TBLK, _W, _C), lambda b, h: (b, h, 0, 0)),
        out_shape=jax.ShapeDtypeStruct((_B, _H, _W, _C), jnp.bfloat16),
    )(data)
    return out.reshape(_B * _H * _W, _C)


@jax.jit
def _dpool(table, roisf, offf):
    mesh = plsc.VectorSubcoreMesh(core_axis_name="c", subcore_axis_name="s")
    run = functools.partial(
        pl.kernel,
        mesh=mesh,
        compiler_params=pltpu.CompilerParams(
            needs_layout_passes=False, use_tc_tiling_on_sc=False),
        out_type=jax.ShapeDtypeStruct((_N * _C * _CELLS,), jnp.float32),
        scratch_types=[
            pltpu.VMEM((5, _L), jnp.float32),
            pltpu.VMEM((_RPW * 2 * _CELLS,), jnp.float32),
            pltpu.VMEM((_HROWS,), jnp.float32),
            pltpu.VMEM((_HROWS,), jnp.float32),
            pltpu.VMEM((_HROWS,), jnp.int32),
            pltpu.VMEM((_HROWS,), jnp.int32),
            pltpu.VMEM((_L,), jnp.float32),
            pltpu.VMEM((_RPW, _C), jnp.float32),
            pltpu.VMEM((_HROWS, _C), jnp.bfloat16),
            pltpu.VMEM((_HROWS, _C), jnp.bfloat16),
            pltpu.VMEM((_RPW * _C * _CELLS,), jnp.float32),
            pltpu.SemaphoreType.DMA,
            pltpu.SemaphoreType.DMA,
        ],
    )(_dpool_body)
    return run(table, roisf, offf)


def kernel(data, rois, offset):
    table = _to_table(data)
    roisf = jnp.transpose(rois, (1, 0)).reshape(-1)
    offf = offset.reshape(-1)
    out = _dpool(table, roisf, offf)
    return out.reshape(_N, _C, _P, _P)


# speedup vs baseline: 1.1236x; 1.0405x over previous
import functools

import jax
import jax.numpy as jnp
from jax import lax
from jax.experimental import pallas as pl
from jax.experimental.pallas import tpu as pltpu
from jax.experimental.pallas import tpu_sc as plsc

_SPATIAL_SCALE = 0.125
_P = 7
_TRANS_STD = 0.1
_B, _C, _H, _W = 2, 64, 256, 256
_N = 512
_NC, _NS, _L = 2, 16, 16
_NW = _NC * _NS
_RPW = _N // _NW
_CELLS = _P * _P
_HROWS = 8 * 4 * _L
_GCHUNK = 128


def _round_half_even(x):
    ri = x.astype(jnp.int32)
    rf = ri.astype(jnp.float32)
    frac = x - rf
    odd = (ri & 1) == 1
    up = (frac > 0.5) | ((frac == 0.5) & odd)
    return jnp.where(up, rf + 1.0, rf)


def _dpool_body(table, roisf, offf, out, rvm, offvm, wb0, wb1, idxb0, idxb1,
                invbuf, acc_blk, rows0, rows1, out_stage, sem0, sem1):
    cid = lax.axis_index("c")
    sid = lax.axis_index("s")
    wid = sid * _NC + cid
    rbase = wid * _RPW

    for j in range(5):
        pltpu.sync_copy(roisf.at[pl.ds(j * _N + rbase, _L)], rvm.at[j])
    pltpu.sync_copy(offf.at[pl.ds(rbase * 2 * _CELLS, _RPW * 2 * _CELLS)], offvm)

    iota16 = lax.broadcasted_iota(jnp.int32, (_L,), 0)
    off_row0 = iota16 * (2 * _CELLS)

    rb = rvm[0].astype(jnp.int32)
    x1 = _round_half_even(rvm[1])
    y1 = _round_half_even(rvm[2])
    x2 = _round_half_even(rvm[3])
    y2 = _round_half_even(rvm[4])
    rsw = x1 * _SPATIAL_SCALE - 0.5
    rsh = y1 * _SPATIAL_SCALE - 0.5
    rew = (x2 + 1.0) * _SPATIAL_SCALE - 0.5
    reh = (y2 + 1.0) * _SPATIAL_SCALE - 0.5
    roi_w = jnp.maximum(rew - rsw, 0.1)
    roi_h = jnp.maximum(reh - rsh, 0.1)
    bin_w = roi_w / float(_P)
    bin_h = roi_h / float(_P)
    sub_w = bin_w / 4.0
    sub_h = bin_h / 4.0
    base_i = rb * (_H * _W)

    scat_cols = [(jnp.int32((j // 2) * 32 + (j % 2)) + iota16 * 2) * _CELLS
                 for j in range(4)]

    def coords(cell, half, idxb, wbufr):
        ph = cell // _P
        pw = cell - ph * _P
        csplat = jnp.full((_L,), cell, jnp.int32)
        tx = plsc.load_gather(offvm, [off_row0 + csplat]) * _TRANS_STD
        ty = plsc.load_gather(offvm, [off_row0 + (csplat + _CELLS)]) * _TRANS_STD
        hst = ph.astype(jnp.float32) * bin_h + rsh + ty * roi_h
        wst = pw.astype(jnp.float32) * bin_w + rsw + tx * roi_w
        cnt = jnp.zeros((_L,), jnp.float32)
        for sl in range(8):
            s = half * 8 + sl
            ih, iw = s // 4, s % 4
            w_ = wst + float(iw) * sub_w
            h_ = hst + float(ih) * sub_h
            mask = (w_ > -0.5) & (w_ < _W - 0.5) & (h_ > -0.5) & (h_ < _H - 0.5)
            mf = jnp.where(mask, 1.0, 0.0)
            cnt = cnt + mf
            wc = jnp.clip(w_, 0.0, _W - 1.0)
            hc = jnp.clip(h_, 0.0, _H - 1.0)
            w0 = wc.astype(jnp.int32)
            h0 = hc.astype(jnp.int32)
            w1 = jnp.minimum(w0 + 1, _W - 1)
            h1 = jnp.minimum(h0 + 1, _H - 1)
            lw = wc - w0.astype(jnp.float32)
            lh = hc - h0.astype(jnp.float32)
            hw = 1.0 - lw
            hh = 1.0 - lh
            r0 = base_i + h0 * _W
            r1 = base_i + h1 * _W
            idxs = (r0 + w0, r0 + w1, r1 + w0, r1 + w1)
            ws = (hh * hw * mf, hh * lw * mf, lh * hw * mf, lh * lw * mf)
            for ci in range(4):
                k = sl * 4 + ci
                idxb[pl.ds(k * _L, _L)] = idxs[ci]
                wbufr[pl.ds(k * _L, _L)] = ws[ci]
        return cnt

    def fire(idxb, rows, sem):
        for g in range(_HROWS // _GCHUNK):
            pltpu.async_copy(
                table.at[idxb.at[pl.ds(g * _GCHUNK, _GCHUNK)]],
                rows.at[pl.ds(g * _GCHUNK, _GCHUNK)], sem)

    def drain(rows, sem):
        pltpu.make_async_copy(table.at[pl.ds(0, _HROWS)], rows, sem).wait()

    ksplats = [jnp.full((_L,), kk, jnp.int32) for kk in range(16)]

    def accum(rows, wbufr, r, rsplat, acc):
        w_lo = plsc.load_gather(wbufr, [iota16 * _L + rsplat])
        w_hi = plsc.load_gather(wbufr, [iota16 * _L + rsplat + 16 * _L])
        for g in range(4):
            a = [jnp.zeros((32,), jnp.bfloat16) for _ in range(2)]
            for kk in range(8):
                k = g * 8 + kk
                wsrc = w_lo if k < 16 else w_hi
                wv = jnp.take_along_axis(wsrc, ksplats[k % 16], axis=0)
                wb16 = plsc.pack(wv, wv, format=plsc.PackFormat.INTERLEAVED)
                row = k * _L + r
                for h in range(2):
                    a[h] = a[h] + wb16 * rows[row, pl.ds(h * 32, 32)]
            for h in range(2):
                ev, od = plsc.unpack(a[h], format=plsc.PackFormat.INTERLEAVED)
                acc[2 * h] = acc[2 * h] + ev
                acc[2 * h + 1] = acc[2 * h + 1] + od
        return acc

    def accum_h0(rh, _):
        for u in range(2):
            r = rh * 2 + u
            rsplat = jnp.full((_L,), r, jnp.int32)
            acc = [jnp.zeros((_L,), jnp.float32) for _ in range(4)]
            acc = accum(rows0, wb0, r, rsplat, acc)
            for j in range(4):
                acc_blk[r, pl.ds(j * _L, _L)] = acc[j]
        return 0

    def make_accum_h1(cell):
        def accum_h1(rh, _):
            for u in range(2):
                r = rh * 2 + u
                rsplat = jnp.full((_L,), r, jnp.int32)
                invs = plsc.load_gather(invbuf, [rsplat])
                acc = [acc_blk[r, pl.ds(j * _L, _L)] for j in range(4)]
                acc = accum(rows1, wb1, r, rsplat, acc)
                sbase = jnp.full((_L,), r * (_C * _CELLS) + cell, jnp.int32)
                for j in range(4):
                    plsc.store_scatter(out_stage, [sbase + scat_cols[j]],
                                       acc[j] * invs)
            return 0
        return accum_h1

    cnt0_init = coords(jnp.int32(0), 0, idxb0, wb0)
    fire(idxb0, rows0, sem0)

    def cell_body(cell, cnt0):
        cnt1 = coords(cell, 1, idxb1, wb1)
        fire(idxb1, rows1, sem1)
        invbuf[...] = 1.0 / jnp.maximum(cnt0 + cnt1, 1.0)
        drain(rows0, sem0)
        lax.fori_loop(0, _RPW // 2, accum_h0, 0)
        cnext = jnp.minimum(cell + 1, _CELLS - 1)
        cnt0_new = coords(cnext, 0, idxb0, wb0)

        @pl.when(cell < _CELLS - 1)
        def _():
            fire(idxb0, rows0, sem0)

        drain(rows1, sem1)
        lax.fori_loop(0, _RPW // 2, make_accum_h1(cell), 0)
        return cnt0_new

    lax.fori_loop(0, _CELLS, cell_body, cnt0_init)
    pltpu.sync_copy(out_stage,
                    out.at[pl.ds(rbase * _C * _CELLS, _RPW * _C * _CELLS)])


_TBLK = 8


def _to_table(data):
    return jnp.transpose(data, (0, 2, 3, 1)).reshape(
        _B * _H * _W, _C).astype(jnp.bfloat16)


@jax.jit
def _dpool(table, roisf, offf):
    mesh = plsc.VectorSubcoreMesh(core_axis_name="c", subcore_axis_name="s")
    run = functools.partial(
        pl.kernel,
        mesh=mesh,
        compiler_params=pltpu.CompilerParams(
            needs_layout_passes=False, use_tc_tiling_on_sc=False),
        out_type=jax.ShapeDtypeStruct((_N * _C * _CELLS,), jnp.float32),
        scratch_types=[
            pltpu.VMEM((5, _L), jnp.float32),
            pltpu.VMEM((_RPW * 2 * _CELLS,), jnp.float32),
            pltpu.VMEM((_HROWS,), jnp.float32),
            pltpu.VMEM((_HROWS,), jnp.float32),
            pltpu.VMEM((_HROWS,), jnp.int32),
            pltpu.VMEM((_HROWS,), jnp.int32),
            pltpu.VMEM((_L,), jnp.float32),
            pltpu.VMEM((_RPW, _C), jnp.float32),
            pltpu.VMEM((_HROWS, _C), jnp.bfloat16),
            pltpu.VMEM((_HROWS, _C), jnp.bfloat16),
            pltpu.VMEM((_RPW * _C * _CELLS,), jnp.float32),
            pltpu.SemaphoreType.DMA,
            pltpu.SemaphoreType.DMA,
        ],
    )(_dpool_body)
    return run(table, roisf, offf)


def kernel(data, rois, offset):
    table = _to_table(data)
    roisf = jnp.transpose(rois, (1, 0)).reshape(-1)
    offf = offset.reshape(-1)
    out = _dpool(table, roisf, offf)
    return out.reshape(_N, _C, _P, _P)
